# baseline (device time: 70925 ns/iter reference)
import numpy as np

import jax
import jax.numpy as jnp
from jax import lax
from jax.experimental import pallas as pl
from jax.experimental.pallas import tpu as pltpu

N_DEV = 4
SQ = 1024
SKV = 1024
HQ_LOCAL = 8
DH = 128
BLK = 64
SCALE = 0.08838834764831843
LOG2E = 1.4426950408889634
CH = SQ // N_DEV
HN = SQ // 2

_BF16 = jnp.bfloat16
_F32 = jnp.float32

_qb = np.arange(SQ) // BLK
_kb = np.arange(SKV) // BLK
_MASK = (
    (_qb[:, None] == _kb[None, :])
    | (_kb[None, :] == 0)
    | ((_qb[:, None] + _kb[None, :]) % 3 == 0)
)
_BIAS = np.where(_MASK, 0.0, -1e9).astype(np.float32)


def _compute_chunk(c, x_ref, wq_ref, kb16, vb16, wo_ref, bias_ref):
    xc = x_ref[0, pl.ds(c * CH, CH), :].astype(_BF16)
    qc = jnp.dot(xc, wq_ref[...], preferred_element_type=_F32).astype(_BF16)
    biasc = bias_ref[pl.ds(c * CH, CH), :]

    ctx_parts = []
    for h in range(HQ_LOCAL):
        qh = qc[:, h * DH:(h + 1) * DH]
        kh = kb16[:, h, :]
        sc = lax.dot_general(
            qh, kh, (((1,), (1,)), ((), ())), preferred_element_type=_F32
        )
        e = jnp.exp2(sc + biasc)
        recip = 1.0 / jnp.sum(e, axis=-1, keepdims=True)
        av = jnp.dot(e.astype(_BF16), vb16[:, h, :],
                     preferred_element_type=_F32)
        ctx_parts.append(av * recip)
    ctx = jnp.concatenate(ctx_parts, axis=1).astype(_BF16)
    return jnp.dot(ctx, wo_ref[...], preferred_element_type=_F32)


def _body(x_ref, wq_ref, k_ref, v_ref, wo_ref, bias_ref, out_ref,
          scw, rcw, sccw, rccw, owna, ownb, agr, kb16, vb16,
          rs_send, rs_recv, ag_send, ag_recv):
    my = lax.axis_index("i")
    left = lax.rem(my + (N_DEV - 1), N_DEV)
    right = lax.rem(my + 1, N_DEV)

    kb16[...] = k_ref[0].astype(_BF16)
    vb16[...] = v_ref[0].astype(_BF16)

    barrier_sem = pltpu.get_barrier_semaphore()
    for nbr in (left, right):
        pl.semaphore_signal(
            barrier_sem, inc=1,
            device_id=(nbr,), device_id_type=pl.DeviceIdType.MESH,
        )
    pl.semaphore_wait(barrier_sem, 2)

    def rs_pair(s):
        cw = pltpu.make_async_remote_copy(
            src_ref=scw.at[s], dst_ref=rcw.at[s],
            send_sem=rs_send.at[0, s], recv_sem=rs_recv.at[0, s],
            device_id=(right,), device_id_type=pl.DeviceIdType.MESH,
        )
        ccw = pltpu.make_async_remote_copy(
            src_ref=sccw.at[s], dst_ref=rccw.at[s],
            send_sem=rs_send.at[1, s], recv_sem=rs_recv.at[1, s],
            device_id=(left,), device_id_type=pl.DeviceIdType.MESH,
        )
        return cw, ccw

    args = (x_ref, wq_ref, kb16, vb16, wo_ref, bias_ref)

    c0 = _compute_chunk(my, *args)
    scw[0] = c0[:, :HN].astype(_BF16)
    sccw[0] = c0[:, HN:].astype(_BF16)
    cw0, ccw0 = rs_pair(0)
    cw0.start()
    ccw0.start()

    c_cwl = _compute_chunk(lax.rem(my + 3, N_DEV), *args)
    c_ccwl = _compute_chunk(lax.rem(my + 1, N_DEV), *args)
    cw0.wait()
    ccw0.wait()
    scw[1] = (c_cwl[:, :HN] + rcw[0].astype(_F32)).astype(_BF16)
    sccw[1] = (c_ccwl[:, HN:] + rccw[0].astype(_F32)).astype(_BF16)
    cw1, ccw1 = rs_pair(1)
    cw1.start()
    ccw1.start()

    c2 = _compute_chunk(lax.rem(my + 2, N_DEV), *args)
    cw1.wait()
    ccw1.wait()
    scw[2] = (c2[:, :HN] + rcw[1].astype(_F32)).astype(_BF16)
    sccw[2] = (c2[:, HN:] + rccw[1].astype(_F32)).astype(_BF16)
    cw2, ccw2 = rs_pair(2)
    cw2.start()
    ccw2.start()
    cw2.wait()
    ccw2.wait()

    owned_a = rcw[2].astype(_F32) + c_ccwl[:, :HN]
    owned_b = rccw[2].astype(_F32) + c_cwl[:, HN:]
    owna[...] = owned_a.astype(_BF16)
    ownb[...] = owned_b.astype(_BF16)

    def ag(src, slot, dev):
        return pltpu.make_async_remote_copy(
            src_ref=src, dst_ref=agr.at[slot],
            send_sem=ag_send.at[slot], recv_sem=ag_recv.at[slot],
            device_id=(dev,), device_id_type=pl.DeviceIdType.MESH,
        )

    d_ar = ag(owna, 0, right)
    d_al = ag(owna, 1, left)
    d_br = ag(ownb, 2, right)
    d_bl = ag(ownb, 3, left)
    d_ar.start()
    d_al.start()
    d_br.start()
    d_bl.start()

    out_ref[0, pl.ds(lax.rem(my + 1, N_DEV) * CH, CH), 0:HN] = owned_a
    out_ref[0, pl.ds(lax.rem(my + 3, N_DEV) * CH, CH), HN:SQ] = owned_b

    d_ar.wait_recv()
    d_fa = ag(agr.at[0], 4, right)
    d_fa.start()
    d_bl.wait_recv()
    d_fb = ag(agr.at[3], 5, left)
    d_fb.start()

    out_ref[0, pl.ds(my * CH, CH), 0:HN] = agr[0].astype(_F32)
    out_ref[0, pl.ds(my * CH, CH), HN:SQ] = agr[3].astype(_F32)

    d_al.wait_recv()
    d_br.wait_recv()
    r2 = lax.rem(my + 2, N_DEV)
    out_ref[0, pl.ds(r2 * CH, CH), 0:HN] = agr[1].astype(_F32)
    out_ref[0, pl.ds(r2 * CH, CH), HN:SQ] = agr[2].astype(_F32)

    d_fa.wait_recv()
    d_fb.wait_recv()
    out_ref[0, pl.ds(lax.rem(my + 3, N_DEV) * CH, CH), 0:HN] = (
        agr[4].astype(_F32))
    out_ref[0, pl.ds(lax.rem(my + 1, N_DEV) * CH, CH), HN:SQ] = (
        agr[5].astype(_F32))

    for d in (d_ar, d_al, d_br, d_bl, d_fa, d_fb):
        d.wait_send()


def kernel(x, Wq, K_ext, V_ext, Wo):
    my = lax.axis_index("i")
    Wq_l = (
        lax.dynamic_slice(Wq, (0, my * (HQ_LOCAL * DH)), (SQ, HQ_LOCAL * DH))
        * (SCALE * LOG2E)
    ).astype(_BF16)
    Wo_l = lax.dynamic_slice(
        Wo, (my * (HQ_LOCAL * DH), 0), (HQ_LOCAL * DH, SQ)
    ).astype(_BF16)
    Kb = K_ext.reshape(1, SKV, HQ_LOCAL, DH)
    Vb = V_ext.reshape(1, SKV, HQ_LOCAL, DH)
    bias = jnp.asarray(_BIAS)

    out = pl.pallas_call(
        _body,
        out_shape=jax.ShapeDtypeStruct((1, SQ, SQ), _F32),
        in_specs=[pl.BlockSpec(memory_space=pltpu.VMEM)] * 6,
        out_specs=pl.BlockSpec(memory_space=pltpu.VMEM),
        scratch_shapes=[
            pltpu.VMEM((N_DEV - 1, CH, HN), _BF16),
            pltpu.VMEM((N_DEV - 1, CH, HN), _BF16),
            pltpu.VMEM((N_DEV - 1, CH, HN), _BF16),
            pltpu.VMEM((N_DEV - 1, CH, HN), _BF16),
            pltpu.VMEM((CH, HN), _BF16),
            pltpu.VMEM((CH, HN), _BF16),
            pltpu.VMEM((6, CH, HN), _BF16),
            pltpu.VMEM((SKV, HQ_LOCAL, DH), _BF16),
            pltpu.VMEM((SKV, HQ_LOCAL, DH), _BF16),
            pltpu.SemaphoreType.DMA((2, N_DEV - 1)),
            pltpu.SemaphoreType.DMA((2, N_DEV - 1)),
            pltpu.SemaphoreType.DMA((6,)),
            pltpu.SemaphoreType.DMA((6,)),
        ],
        compiler_params=pltpu.CompilerParams(collective_id=0),
    )(x, Wq_l, Kb, Vb, Wo_l, bias)
    return out


# device time: 50365 ns/iter; 1.4082x vs baseline; 1.4082x over previous
import numpy as np

import jax
import jax.numpy as jnp
from jax import lax
from jax.experimental import pallas as pl
from jax.experimental.pallas import tpu as pltpu

N_DEV = 4
SQ = 1024
SKV = 1024
HQ_LOCAL = 8
DH = 128
BLK = 64
SCALE = 0.08838834764831843
LOG2E = 1.4426950408889634
CH = SQ // N_DEV
HN = SQ // 2

_BF16 = jnp.bfloat16
_F32 = jnp.float32

_qb = np.arange(SQ) // BLK
_kb = np.arange(SKV) // BLK
_MASK = (
    (_qb[:, None] == _kb[None, :])
    | (_kb[None, :] == 0)
    | ((_qb[:, None] + _kb[None, :]) % 3 == 0)
)
_BIAS = np.where(_MASK, 0.0, -1e9).astype(np.float32)


def _compute_chunk(c, x_ref, wq_ref, kb16, vb16, wo_ref, bias_ref):
    xc = x_ref[pl.ds(c * CH, CH), :]
    qc = jnp.dot(xc, wq_ref[...], preferred_element_type=_F32).astype(_BF16)
    biasc = bias_ref[pl.ds(c * CH, CH), :]

    ctx_parts = []
    for h in range(HQ_LOCAL):
        qh = qc[:, h * DH:(h + 1) * DH]
        kh = kb16[:, h * DH:(h + 1) * DH]
        sc = lax.dot_general(
            qh, kh, (((1,), (1,)), ((), ())), preferred_element_type=_F32
        )
        e = jnp.exp2(sc + biasc)
        recip = 1.0 / jnp.sum(e, axis=-1, keepdims=True)
        av = jnp.dot(e.astype(_BF16), vb16[:, h * DH:(h + 1) * DH],
                     preferred_element_type=_F32)
        ctx_parts.append(av * recip)
    ctx = jnp.concatenate(ctx_parts, axis=1).astype(_BF16)
    return jnp.dot(ctx, wo_ref[...], preferred_element_type=_F32)


def _body(x_ref, wq_ref, kb16, vb16, wo_ref, bias_ref, out_ref,
          scw, rcw, sccw, rccw, owna, ownb, agr,
          rs_send, rs_recv, ag_send, ag_recv):
    my = lax.axis_index("i")
    left = lax.rem(my + (N_DEV - 1), N_DEV)
    right = lax.rem(my + 1, N_DEV)

    barrier_sem = pltpu.get_barrier_semaphore()
    for nbr in (left, right):
        pl.semaphore_signal(
            barrier_sem, inc=1,
            device_id=(nbr,), device_id_type=pl.DeviceIdType.MESH,
        )
    pl.semaphore_wait(barrier_sem, 2)

    def rs_pair(s):
        cw = pltpu.make_async_remote_copy(
            src_ref=scw.at[s], dst_ref=rcw.at[s],
            send_sem=rs_send.at[0, s], recv_sem=rs_recv.at[0, s],
            device_id=(right,), device_id_type=pl.DeviceIdType.MESH,
        )
        ccw = pltpu.make_async_remote_copy(
            src_ref=sccw.at[s], dst_ref=rccw.at[s],
            send_sem=rs_send.at[1, s], recv_sem=rs_recv.at[1, s],
            device_id=(left,), device_id_type=pl.DeviceIdType.MESH,
        )
        return cw, ccw

    args = (x_ref, wq_ref, kb16, vb16, wo_ref, bias_ref)

    c0 = _compute_chunk(my, *args)
    scw[0] = c0[:, :HN].astype(_BF16)
    sccw[0] = c0[:, HN:].astype(_BF16)
    cw0, ccw0 = rs_pair(0)
    cw0.start()
    ccw0.start()

    c_cwl = _compute_chunk(lax.rem(my + 3, N_DEV), *args)
    c_ccwl = _compute_chunk(lax.rem(my + 1, N_DEV), *args)
    cw0.wait()
    ccw0.wait()
    scw[1] = (c_cwl[:, :HN] + rcw[0].astype(_F32)).astype(_BF16)
    sccw[1] = (c_ccwl[:, HN:] + rccw[0].astype(_F32)).astype(_BF16)
    cw1, ccw1 = rs_pair(1)
    cw1.start()
    ccw1.start()

    c2 = _compute_chunk(lax.rem(my + 2, N_DEV), *args)
    cw1.wait()
    ccw1.wait()
    scw[2] = (c2[:, :HN] + rcw[1].astype(_F32)).astype(_BF16)
    sccw[2] = (c2[:, HN:] + rccw[1].astype(_F32)).astype(_BF16)
    cw2, ccw2 = rs_pair(2)
    cw2.start()
    ccw2.start()
    cw2.wait()
    ccw2.wait()

    owned_a = rcw[2].astype(_F32) + c_ccwl[:, :HN]
    owned_b = rccw[2].astype(_F32) + c_cwl[:, HN:]
    owna[...] = owned_a.astype(_BF16)
    ownb[...] = owned_b.astype(_BF16)

    def ag(src, slot, dev):
        return pltpu.make_async_remote_copy(
            src_ref=src, dst_ref=agr.at[slot],
            send_sem=ag_send.at[slot], recv_sem=ag_recv.at[slot],
            device_id=(dev,), device_id_type=pl.DeviceIdType.MESH,
        )

    d_ar = ag(owna, 0, right)
    d_al = ag(owna, 1, left)
    d_br = ag(ownb, 2, right)
    d_bl = ag(ownb, 3, left)
    d_ar.start()
    d_al.start()
    d_br.start()
    d_bl.start()

    out_ref[0, pl.ds(lax.rem(my + 1, N_DEV) * CH, CH), 0:HN] = owned_a
    out_ref[0, pl.ds(lax.rem(my + 3, N_DEV) * CH, CH), HN:SQ] = owned_b

    d_ar.wait_recv()
    d_fa = ag(agr.at[0], 4, right)
    d_fa.start()
    d_bl.wait_recv()
    d_fb = ag(agr.at[3], 5, left)
    d_fb.start()

    out_ref[0, pl.ds(my * CH, CH), 0:HN] = agr[0].astype(_F32)
    out_ref[0, pl.ds(my * CH, CH), HN:SQ] = agr[3].astype(_F32)

    d_al.wait_recv()
    d_br.wait_recv()
    r2 = lax.rem(my + 2, N_DEV)
    out_ref[0, pl.ds(r2 * CH, CH), 0:HN] = agr[1].astype(_F32)
    out_ref[0, pl.ds(r2 * CH, CH), HN:SQ] = agr[2].astype(_F32)

    d_fa.wait_recv()
    d_fb.wait_recv()
    out_ref[0, pl.ds(lax.rem(my + 3, N_DEV) * CH, CH), 0:HN] = (
        agr[4].astype(_F32))
    out_ref[0, pl.ds(lax.rem(my + 1, N_DEV) * CH, CH), HN:SQ] = (
        agr[5].astype(_F32))

    for d in (d_ar, d_al, d_br, d_bl, d_fa, d_fb):
        d.wait_send()


def kernel(x, Wq, K_ext, V_ext, Wo):
    my = lax.axis_index("i")
    Wq_l = (
        lax.dynamic_slice(Wq, (0, my * (HQ_LOCAL * DH)), (SQ, HQ_LOCAL * DH))
        * (SCALE * LOG2E)
    ).astype(_BF16)
    Wo_l = lax.dynamic_slice(
        Wo, (my * (HQ_LOCAL * DH), 0), (HQ_LOCAL * DH, SQ)
    ).astype(_BF16)
    xb = x[0].astype(_BF16)
    Kb = K_ext[0].astype(_BF16).reshape(SKV, HQ_LOCAL * DH)
    Vb = V_ext[0].astype(_BF16).reshape(SKV, HQ_LOCAL * DH)
    bias = jnp.asarray(_BIAS)

    out = pl.pallas_call(
        _body,
        out_shape=jax.ShapeDtypeStruct((1, SQ, SQ), _F32),
        in_specs=[pl.BlockSpec(memory_space=pltpu.VMEM)] * 6,
        out_specs=pl.BlockSpec(memory_space=pltpu.VMEM),
        scratch_shapes=[
            pltpu.VMEM((N_DEV - 1, CH, HN), _BF16),
            pltpu.VMEM((N_DEV - 1, CH, HN), _BF16),
            pltpu.VMEM((N_DEV - 1, CH, HN), _BF16),
            pltpu.VMEM((N_DEV - 1, CH, HN), _BF16),
            pltpu.VMEM((CH, HN), _BF16),
            pltpu.VMEM((CH, HN), _BF16),
            pltpu.VMEM((6, CH, HN), _BF16),
            pltpu.SemaphoreType.DMA((2, N_DEV - 1)),
            pltpu.SemaphoreType.DMA((2, N_DEV - 1)),
            pltpu.SemaphoreType.DMA((6,)),
            pltpu.SemaphoreType.DMA((6,)),
        ],
        compiler_params=pltpu.CompilerParams(collective_id=0),
    )(xb, Wq_l, Kb, Vb, Wo_l, bias)
    return out


# device time: 50070 ns/iter; 1.4165x vs baseline; 1.0059x over previous
import numpy as np

import jax
import jax.numpy as jnp
from jax import lax
from jax.experimental import pallas as pl
from jax.experimental.pallas import tpu as pltpu

N_DEV = 4
SQ = 1024
SKV = 1024
HQ_LOCAL = 8
DH = 128
BLK = 64
SCALE = 0.08838834764831843
LOG2E = 1.4426950408889634
CH = SQ // N_DEV
HN = SQ // 2

_BF16 = jnp.bfloat16
_F32 = jnp.float32

_qb = np.arange(SQ) // BLK
_kb = np.arange(SKV) // BLK
_MASK = (
    (_qb[:, None] == _kb[None, :])
    | (_kb[None, :] == 0)
    | ((_qb[:, None] + _kb[None, :]) % 3 == 0)
)
_BIAS = np.where(_MASK, 0.0, -1e9).astype(np.float32)


def _compute_chunk(c, x_ref, wq_ref, kb16, vb16, wo_ref, bias_ref):
    xc = x_ref[pl.ds(c * CH, CH), :]
    qc = jnp.dot(xc, wq_ref[...], preferred_element_type=_F32).astype(_BF16)
    biasc = bias_ref[pl.ds(c * CH, CH), :]

    ctx_parts = []
    for h in range(HQ_LOCAL):
        qh = qc[:, h * DH:(h + 1) * DH]
        kh = kb16[:, h * DH:(h + 1) * DH]
        sc = lax.dot_general(
            qh, kh, (((1,), (1,)), ((), ())), preferred_element_type=_F32
        )
        e = jnp.exp2((sc + biasc).astype(_BF16))
        recip = 1.0 / jnp.sum(e, axis=-1, keepdims=True, dtype=_F32)
        av = jnp.dot(e, vb16[:, h * DH:(h + 1) * DH],
                     preferred_element_type=_F32)
        ctx_parts.append(av * recip)
    ctx = jnp.concatenate(ctx_parts, axis=1).astype(_BF16)
    return jnp.dot(ctx, wo_ref[...], preferred_element_type=_F32)


def _body(x_ref, wq_ref, kb16, vb16, wo_ref, bias_ref, out_ref,
          scw, rcw, sccw, rccw, owna, ownb, agr,
          rs_send, rs_recv, ag_send, ag_recv):
    my = lax.axis_index("i")
    left = lax.rem(my + (N_DEV - 1), N_DEV)
    right = lax.rem(my + 1, N_DEV)

    barrier_sem = pltpu.get_barrier_semaphore()
    for nbr in (left, right):
        pl.semaphore_signal(
            barrier_sem, inc=1,
            device_id=(nbr,), device_id_type=pl.DeviceIdType.MESH,
        )
    pl.semaphore_wait(barrier_sem, 2)

    def rs_pair(s):
        cw = pltpu.make_async_remote_copy(
            src_ref=scw.at[s], dst_ref=rcw.at[s],
            send_sem=rs_send.at[0, s], recv_sem=rs_recv.at[0, s],
            device_id=(right,), device_id_type=pl.DeviceIdType.MESH,
        )
        ccw = pltpu.make_async_remote_copy(
            src_ref=sccw.at[s], dst_ref=rccw.at[s],
            send_sem=rs_send.at[1, s], recv_sem=rs_recv.at[1, s],
            device_id=(left,), device_id_type=pl.DeviceIdType.MESH,
        )
        return cw, ccw

    args = (x_ref, wq_ref, kb16, vb16, wo_ref, bias_ref)

    c0 = _compute_chunk(my, *args)
    scw[0] = c0[:, :HN].astype(_BF16)
    sccw[0] = c0[:, HN:].astype(_BF16)
    cw0, ccw0 = rs_pair(0)
    cw0.start()
    ccw0.start()

    c_cwl = _compute_chunk(lax.rem(my + 3, N_DEV), *args)
    c_ccwl = _compute_chunk(lax.rem(my + 1, N_DEV), *args)
    cw0.wait()
    ccw0.wait()
    scw[1] = (c_cwl[:, :HN] + rcw[0].astype(_F32)).astype(_BF16)
    sccw[1] = (c_ccwl[:, HN:] + rccw[0].astype(_F32)).astype(_BF16)
    cw1, ccw1 = rs_pair(1)
    cw1.start()
    ccw1.start()

    c2 = _compute_chunk(lax.rem(my + 2, N_DEV), *args)
    cw1.wait()
    ccw1.wait()
    scw[2] = (c2[:, :HN] + rcw[1].astype(_F32)).astype(_BF16)
    sccw[2] = (c2[:, HN:] + rccw[1].astype(_F32)).astype(_BF16)
    cw2, ccw2 = rs_pair(2)
    cw2.start()
    ccw2.start()
    cw2.wait()
    ccw2.wait()

    owned_a = rcw[2].astype(_F32) + c_ccwl[:, :HN]
    owned_b = rccw[2].astype(_F32) + c_cwl[:, HN:]
    owna[...] = owned_a.astype(_BF16)
    ownb[...] = owned_b.astype(_BF16)

    def ag(src, slot, dev):
        return pltpu.make_async_remote_copy(
            src_ref=src, dst_ref=agr.at[slot],
            send_sem=ag_send.at[slot], recv_sem=ag_recv.at[slot],
            device_id=(dev,), device_id_type=pl.DeviceIdType.MESH,
        )

    d_ar = ag(owna, 0, right)
    d_al = ag(owna, 1, left)
    d_br = ag(ownb, 2, right)
    d_bl = ag(ownb, 3, left)
    d_ar.start()
    d_al.start()
    d_br.start()
    d_bl.start()

    out_ref[0, pl.ds(lax.rem(my + 1, N_DEV) * CH, CH), 0:HN] = owned_a
    out_ref[0, pl.ds(lax.rem(my + 3, N_DEV) * CH, CH), HN:SQ] = owned_b

    d_ar.wait_recv()
    d_fa = ag(agr.at[0], 4, right)
    d_fa.start()
    d_bl.wait_recv()
    d_fb = ag(agr.at[3], 5, left)
    d_fb.start()

    out_ref[0, pl.ds(my * CH, CH), 0:HN] = agr[0].astype(_F32)
    out_ref[0, pl.ds(my * CH, CH), HN:SQ] = agr[3].astype(_F32)

    d_al.wait_recv()
    d_br.wait_recv()
    r2 = lax.rem(my + 2, N_DEV)
    out_ref[0, pl.ds(r2 * CH, CH), 0:HN] = agr[1].astype(_F32)
    out_ref[0, pl.ds(r2 * CH, CH), HN:SQ] = agr[2].astype(_F32)

    d_fa.wait_recv()
    d_fb.wait_recv()
    out_ref[0, pl.ds(lax.rem(my + 3, N_DEV) * CH, CH), 0:HN] = (
        agr[4].astype(_F32))
    out_ref[0, pl.ds(lax.rem(my + 1, N_DEV) * CH, CH), HN:SQ] = (
        agr[5].astype(_F32))

    for d in (d_ar, d_al, d_br, d_bl, d_fa, d_fb):
        d.wait_send()


def kernel(x, Wq, K_ext, V_ext, Wo):
    my = lax.axis_index("i")
    Wq_l = (
        lax.dynamic_slice(Wq, (0, my * (HQ_LOCAL * DH)), (SQ, HQ_LOCAL * DH))
        * (SCALE * LOG2E)
    ).astype(_BF16)
    Wo_l = lax.dynamic_slice(
        Wo, (my * (HQ_LOCAL * DH), 0), (HQ_LOCAL * DH, SQ)
    ).astype(_BF16)
    xb = x[0].astype(_BF16)
    Kb = K_ext[0].astype(_BF16).reshape(SKV, HQ_LOCAL * DH)
    Vb = V_ext[0].astype(_BF16).reshape(SKV, HQ_LOCAL * DH)
    bias = jnp.asarray(_BIAS)

    out = pl.pallas_call(
        _body,
        out_shape=jax.ShapeDtypeStruct((1, SQ, SQ), _F32),
        in_specs=[pl.BlockSpec(memory_space=pltpu.VMEM)] * 6,
        out_specs=pl.BlockSpec(memory_space=pltpu.VMEM),
        scratch_shapes=[
            pltpu.VMEM((N_DEV - 1, CH, HN), _BF16),
            pltpu.VMEM((N_DEV - 1, CH, HN), _BF16),
            pltpu.VMEM((N_DEV - 1, CH, HN), _BF16),
            pltpu.VMEM((N_DEV - 1, CH, HN), _BF16),
            pltpu.VMEM((CH, HN), _BF16),
            pltpu.VMEM((CH, HN), _BF16),
            pltpu.VMEM((6, CH, HN), _BF16),
            pltpu.SemaphoreType.DMA((2, N_DEV - 1)),
            pltpu.SemaphoreType.DMA((2, N_DEV - 1)),
            pltpu.SemaphoreType.DMA((6,)),
            pltpu.SemaphoreType.DMA((6,)),
        ],
        compiler_params=pltpu.CompilerParams(collective_id=0),
    )(xb, Wq_l, Kb, Vb, Wo_l, bias)
    return out
